# baseline (speedup 1.0000x reference)
import functools

import jax
import jax.numpy as jnp
from jax import lax
from jax.experimental import pallas as pl
from jax.experimental.pallas import tpu as pltpu
from jax.experimental.pallas import tpu_sc as plsc

N = 10000
E = 320000
D_IN = 128
H1 = 128
H2 = 64
D_OUT = 128

NC = 2
NS = 16
NW = NC * NS
EPW = E // NW
CH = 125
NCHUNK = EPW // CH
N_PAD = 10240
STRIPE = N_PAD // NS
ZCH = 5


BLK = 2000


def _sc_mesh():
    return plsc.VectorSubcoreMesh(core_axis_name="c", subcore_axis_name="s")


def _sc_degree(dst3):
    @functools.partial(
        pl.kernel,
        out_type=jax.ShapeDtypeStruct((NC, N_PAD), jnp.float32),
        mesh=_sc_mesh(),
        scratch_types=[
            pltpu.VMEM((NCHUNK, CH), jnp.int32),
            pltpu.VMEM((128,), jnp.float32),
            pltpu.VMEM((128,), jnp.float32),
            pltpu.VMEM_SHARED((N_PAD,), jnp.float32),
        ],
    )
    def k(dst_hbm, out_hbm, idx_v, ones_v, tmp_v, deg_sh):
        c = lax.axis_index("c")
        s = lax.axis_index("s")
        wid = c * NS + s
        for j in range(8):
            ones_v[pl.ds(j * 16, 16)] = jnp.full((16,), 1.0, jnp.float32)
            tmp_v[pl.ds(j * 16, 16)] = jnp.zeros((16,), jnp.float32)
        base = s * STRIPE
        for t in range(ZCH):
            pltpu.sync_copy(tmp_v, deg_sh.at[pl.ds(base + t * 128, 128)])
        plsc.subcore_barrier()
        pltpu.sync_copy(dst_hbm.at[wid], idx_v)

        def body(ch, carry):
            pltpu.sync_copy(ones_v.at[pl.ds(0, CH)],
                            deg_sh.at[idx_v.at[ch]], add=True)
            return carry

        lax.fori_loop(0, NCHUNK, body, 0)
        plsc.subcore_barrier()
        for t in range(ZCH):
            off = base + t * 128
            pltpu.sync_copy(deg_sh.at[pl.ds(off, 128)],
                            out_hbm.at[c, pl.ds(off, 128)])

    return k(dst3)


def _sc_scatter(hwp, src2, dst3, feat):
    @functools.partial(
        pl.kernel,
        out_type=jax.ShapeDtypeStruct((NC, N_PAD, feat), jnp.float32),
        mesh=_sc_mesh(),
        scratch_types=[
            pltpu.VMEM((NCHUNK, CH), jnp.int32),
            pltpu.VMEM((NCHUNK, CH), jnp.int32),
            pltpu.VMEM((128, feat), jnp.float32),
            pltpu.VMEM_SHARED((N_PAD, feat), jnp.float32),
            pltpu.SemaphoreType.DMA,
        ],
    )
    def k(hwp_hbm, src_hbm, dst_hbm, out_hbm, src_v, dst_v, rows_v,
          acc_sh, sem):
        c = lax.axis_index("c")
        s = lax.axis_index("s")
        wid = c * NS + s

        def zbody(r, carry):
            for j in range(feat // 16):
                rows_v[r, pl.ds(j * 16, 16)] = jnp.zeros((16,), jnp.float32)
            return carry

        lax.fori_loop(0, 128, zbody, 0)
        base = s * STRIPE
        for t in range(ZCH):
            pltpu.sync_copy(rows_v, acc_sh.at[pl.ds(base + t * 128, 128)])
        plsc.subcore_barrier()
        pltpu.sync_copy(src_hbm.at[wid], src_v)
        pltpu.sync_copy(dst_hbm.at[wid], dst_v)

        @pl.when(s % 2 == 1)
        def _():
            pl.delay(1100)

        def body(ch, carry):
            pltpu.async_copy(hwp_hbm.at[src_v.at[ch]],
                             rows_v.at[pl.ds(0, CH)], sem).wait()
            pltpu.sync_copy(rows_v.at[pl.ds(0, CH)],
                            acc_sh.at[dst_v.at[ch]], add=True)
            return carry

        lax.fori_loop(0, NCHUNK, body, 0)
        plsc.subcore_barrier()
        for t in range(ZCH):
            off = base + t * 128
            pltpu.sync_copy(acc_sh.at[pl.ds(off, 128)],
                            out_hbm.at[c, pl.ds(off, 128)])

    return k(hwp, src2, dst3)


def _dot(a, b, dims):
    return lax.dot_general(a, b, (dims, ((), ())),
                           preferred_element_type=jnp.float32,
                           precision=lax.Precision.HIGHEST)


def _tc_pre(x, W1, degp):
    def body(x_ref, w_ref, degp_ref, hwp_ref, dinv_ref):
        deg = degp_ref[:, 0] + degp_ref[:, 1] + 1.0
        dinv = lax.rsqrt(deg)
        hw = _dot(x_ref[...], w_ref[...], ((1,), (0,)))
        hwp_ref[...] = hw * dinv[:, None]
        dinv_ref[...] = dinv[:, None]

    return pl.pallas_call(
        body,
        grid=(N // BLK,),
        in_specs=[
            pl.BlockSpec((BLK, D_IN), lambda i: (i, 0)),
            pl.BlockSpec((D_IN, H1), lambda i: (0, 0)),
            pl.BlockSpec((BLK, NC), lambda i: (i, 0)),
        ],
        out_specs=[
            pl.BlockSpec((BLK, H1), lambda i: (i, 0)),
            pl.BlockSpec((BLK, 1), lambda i: (i, 0)),
        ],
        out_shape=[
            jax.ShapeDtypeStruct((N, H1), jnp.float32),
            jax.ShapeDtypeStruct((N, 1), jnp.float32),
        ],
    )(x, W1, degp)


def _tc_post(acc, hwp, dinv, b2d, Wres, Wnext):
    feat = hwp.shape[1]
    nxt = Wnext.shape[1]

    def body(acc_ref, hwp_ref, dinv_ref, b_ref, wr_ref, wn_ref, out_ref):
        dinv = dinv_ref[...]
        z = (acc_ref[0] + acc_ref[1] + hwp_ref[...]) * dinv + b_ref[...]
        z = jnp.maximum(z, 0.0)
        z = jnp.maximum(z + _dot(z, wr_ref[...], ((1,), (1,))), 0.0)
        out_ref[...] = _dot(z, wn_ref[...], ((1,), (0,))) * dinv

    return pl.pallas_call(
        body,
        grid=(N // BLK,),
        in_specs=[
            pl.BlockSpec((NC, BLK, feat), lambda i: (0, i, 0)),
            pl.BlockSpec((BLK, feat), lambda i: (i, 0)),
            pl.BlockSpec((BLK, 1), lambda i: (i, 0)),
            pl.BlockSpec((1, feat), lambda i: (0, 0)),
            pl.BlockSpec((feat, feat), lambda i: (0, 0)),
            pl.BlockSpec((feat, nxt), lambda i: (0, 0)),
        ],
        out_specs=pl.BlockSpec((BLK, nxt), lambda i: (i, 0)),
        out_shape=jax.ShapeDtypeStruct((N, nxt), jnp.float32),
    )(acc, hwp, dinv, b2d, Wres, Wnext)


def _tc_final(acc, hwp, dinv, b2d, Wres):
    feat = hwp.shape[1]

    def body(acc_ref, hwp_ref, dinv_ref, b_ref, wr_ref, out_ref):
        dinv = dinv_ref[...]
        z = (acc_ref[0] + acc_ref[1] + hwp_ref[...]) * dinv + b_ref[...]
        z = jnp.maximum(z, 0.0)
        out_ref[...] = jnp.maximum(z + _dot(z, wr_ref[...], ((1,), (1,))), 0.0)

    return pl.pallas_call(
        body,
        grid=(N // BLK,),
        in_specs=[
            pl.BlockSpec((NC, BLK, feat), lambda i: (0, i, 0)),
            pl.BlockSpec((BLK, feat), lambda i: (i, 0)),
            pl.BlockSpec((BLK, 1), lambda i: (i, 0)),
            pl.BlockSpec((1, feat), lambda i: (0, 0)),
            pl.BlockSpec((feat, feat), lambda i: (0, 0)),
        ],
        out_specs=pl.BlockSpec((BLK, feat), lambda i: (i, 0)),
        out_shape=jax.ShapeDtypeStruct((N, feat), jnp.float32),
    )(acc, hwp, dinv, b2d, Wres)


def kernel(x, edge_index, W1, b1, W2, W3, b3, W4, W5, b5, W6):
    src3 = edge_index[0].reshape(NW, NCHUNK, CH)
    dst3 = edge_index[1].reshape(NW, NCHUNK, CH)

    W3p = jnp.zeros((H1, 128), jnp.float32).at[:, :H2].set(W3)
    b3p = jnp.zeros((128,), jnp.float32).at[:H2].set(b3)
    W4p = jnp.zeros((128, 128), jnp.float32).at[:H2, :H2].set(W4)
    W5p = jnp.zeros((128, D_OUT), jnp.float32).at[:H2, :].set(W5)

    degp = _sc_degree(dst3)[:, :N].T
    hw1p, dinv = _tc_pre(x, W1, degp)

    acc1 = _sc_scatter(hw1p, src3, dst3, H1)[:, :N]
    hw2p = _tc_post(acc1, hw1p, dinv, b1.reshape(1, H1), W2, W3p)

    acc2 = _sc_scatter(hw2p, src3, dst3, 128)[:, :N]
    hw3p = _tc_post(acc2, hw2p, dinv, b3p.reshape(1, 128), W4p, W5p)

    acc3 = _sc_scatter(hw3p, src3, dst3, D_OUT)[:, :N]
    return _tc_final(acc3, hw3p, dinv, b5.reshape(1, D_OUT), W6)


# speedup vs baseline: 1.0103x; 1.0103x over previous
import functools

import jax
import jax.numpy as jnp
from jax import lax
from jax.experimental import pallas as pl
from jax.experimental.pallas import tpu as pltpu
from jax.experimental.pallas import tpu_sc as plsc

N = 10000
E = 320000
D_IN = 128
H1 = 128
H2 = 64
D_OUT = 128

NC = 2
NS = 16
NW = NC * NS
EPW = E // NW
CH = 125
NCHUNK = EPW // CH
N_PAD = 10240
STRIPE = N_PAD // NS
ZCH = 5


BLK = 2000


def _sc_mesh():
    return plsc.VectorSubcoreMesh(core_axis_name="c", subcore_axis_name="s")


def _sc_degree(dst3):
    @functools.partial(
        pl.kernel,
        out_type=jax.ShapeDtypeStruct((NC, N_PAD), jnp.float32),
        mesh=_sc_mesh(),
        scratch_types=[
            pltpu.VMEM((NCHUNK, CH), jnp.int32),
            pltpu.VMEM((128,), jnp.float32),
            pltpu.VMEM((128,), jnp.float32),
            pltpu.VMEM_SHARED((N_PAD,), jnp.float32),
        ],
    )
    def k(dst_hbm, out_hbm, idx_v, ones_v, tmp_v, deg_sh):
        c = lax.axis_index("c")
        s = lax.axis_index("s")
        wid = c * NS + s
        for j in range(8):
            ones_v[pl.ds(j * 16, 16)] = jnp.full((16,), 1.0, jnp.float32)
            tmp_v[pl.ds(j * 16, 16)] = jnp.zeros((16,), jnp.float32)
        base = s * STRIPE
        for t in range(ZCH):
            pltpu.sync_copy(tmp_v, deg_sh.at[pl.ds(base + t * 128, 128)])
        plsc.subcore_barrier()
        pltpu.sync_copy(dst_hbm.at[wid], idx_v)

        def body(ch, carry):
            pltpu.sync_copy(ones_v.at[pl.ds(0, CH)],
                            deg_sh.at[idx_v.at[ch]], add=True)
            return carry

        lax.fori_loop(0, NCHUNK, body, 0)
        plsc.subcore_barrier()
        for t in range(ZCH):
            off = base + t * 128
            pltpu.sync_copy(deg_sh.at[pl.ds(off, 128)],
                            out_hbm.at[c, pl.ds(off, 128)])

    return k(dst3)


def _sc_scatter(hwp, src3, dst3, feat):
    @functools.partial(
        pl.kernel,
        out_type=jax.ShapeDtypeStruct((NC, N_PAD, feat), jnp.float32),
        mesh=_sc_mesh(),
        scratch_types=[
            pltpu.VMEM((NCHUNK, CH), jnp.int32),
            pltpu.VMEM((NCHUNK, CH), jnp.int32),
            pltpu.VMEM((128, feat), jnp.float32),
            pltpu.VMEM_SHARED((N_PAD, feat), jnp.float32),
            pltpu.SemaphoreType.DMA,
        ],
    )
    def k(hwp_hbm, src_hbm, dst_hbm, out_hbm, src_v, dst_v, rows_v,
          acc_sh, sem):
        c = lax.axis_index("c")
        s = lax.axis_index("s")
        wid = c * NS + s

        def zbody(r, carry):
            for j in range(feat // 16):
                rows_v[r, pl.ds(j * 16, 16)] = jnp.zeros((16,), jnp.float32)
            return carry

        lax.fori_loop(0, 128, zbody, 0)
        base = s * STRIPE
        for t in range(ZCH):
            pltpu.sync_copy(rows_v, acc_sh.at[pl.ds(base + t * 128, 128)])
        plsc.subcore_barrier()
        pltpu.sync_copy(src_hbm.at[wid], src_v)
        pltpu.sync_copy(dst_hbm.at[wid], dst_v)

        def body(ch, carry):
            pltpu.async_copy(hwp_hbm.at[src_v.at[ch]],
                             rows_v.at[pl.ds(0, CH)], sem).wait()
            pltpu.sync_copy(rows_v.at[pl.ds(0, CH)],
                            acc_sh.at[dst_v.at[ch]], add=True)
            return carry

        lax.fori_loop(0, NCHUNK, body, 0)
        plsc.subcore_barrier()
        for t in range(ZCH):
            off = base + t * 128
            pltpu.sync_copy(acc_sh.at[pl.ds(off, 128)],
                            out_hbm.at[c, pl.ds(off, 128)])

    return k(hwp, src3, dst3)


def _dot(a, b, dims):
    return lax.dot_general(a, b, (dims, ((), ())),
                           preferred_element_type=jnp.float32,
                           precision=lax.Precision.HIGHEST)


def _tc_pre(x, W1, degp):
    def body(x_ref, w_ref, degp_ref, hwp_ref, dinv_ref):
        deg = degp_ref[:, 0] + degp_ref[:, 1] + 1.0
        dinv = lax.rsqrt(deg)
        hw = _dot(x_ref[...], w_ref[...], ((1,), (0,)))
        hwp_ref[...] = hw * dinv[:, None]
        dinv_ref[...] = dinv[:, None]

    return pl.pallas_call(
        body,
        grid=(N // BLK,),
        in_specs=[
            pl.BlockSpec((BLK, D_IN), lambda i: (i, 0)),
            pl.BlockSpec((D_IN, H1), lambda i: (0, 0)),
            pl.BlockSpec((BLK, NC), lambda i: (i, 0)),
        ],
        out_specs=[
            pl.BlockSpec((BLK, H1), lambda i: (i, 0)),
            pl.BlockSpec((BLK, 1), lambda i: (i, 0)),
        ],
        out_shape=[
            jax.ShapeDtypeStruct((N, H1), jnp.float32),
            jax.ShapeDtypeStruct((N, 1), jnp.float32),
        ],
    )(x, W1, degp)


def _tc_post(acc, hwp, dinv, b2d, Wres, Wnext):
    feat = hwp.shape[1]
    nxt = Wnext.shape[1]

    def body(acc_ref, hwp_ref, dinv_ref, b_ref, wr_ref, wn_ref, out_ref):
        dinv = dinv_ref[...]
        z = (acc_ref[0] + acc_ref[1] + hwp_ref[...]) * dinv + b_ref[...]
        z = jnp.maximum(z, 0.0)
        z = jnp.maximum(z + _dot(z, wr_ref[...], ((1,), (1,))), 0.0)
        out_ref[...] = _dot(z, wn_ref[...], ((1,), (0,))) * dinv

    return pl.pallas_call(
        body,
        grid=(N // BLK,),
        in_specs=[
            pl.BlockSpec((NC, BLK, feat), lambda i: (0, i, 0)),
            pl.BlockSpec((BLK, feat), lambda i: (i, 0)),
            pl.BlockSpec((BLK, 1), lambda i: (i, 0)),
            pl.BlockSpec((1, feat), lambda i: (0, 0)),
            pl.BlockSpec((feat, feat), lambda i: (0, 0)),
            pl.BlockSpec((feat, nxt), lambda i: (0, 0)),
        ],
        out_specs=pl.BlockSpec((BLK, nxt), lambda i: (i, 0)),
        out_shape=jax.ShapeDtypeStruct((N, nxt), jnp.float32),
    )(acc, hwp, dinv, b2d, Wres, Wnext)


def _tc_final(acc, hwp, dinv, b2d, Wres):
    feat = hwp.shape[1]

    def body(acc_ref, hwp_ref, dinv_ref, b_ref, wr_ref, out_ref):
        dinv = dinv_ref[...]
        z = (acc_ref[0] + acc_ref[1] + hwp_ref[...]) * dinv + b_ref[...]
        z = jnp.maximum(z, 0.0)
        out_ref[...] = jnp.maximum(z + _dot(z, wr_ref[...], ((1,), (1,))), 0.0)

    return pl.pallas_call(
        body,
        grid=(N // BLK,),
        in_specs=[
            pl.BlockSpec((NC, BLK, feat), lambda i: (0, i, 0)),
            pl.BlockSpec((BLK, feat), lambda i: (i, 0)),
            pl.BlockSpec((BLK, 1), lambda i: (i, 0)),
            pl.BlockSpec((1, feat), lambda i: (0, 0)),
            pl.BlockSpec((feat, feat), lambda i: (0, 0)),
        ],
        out_specs=pl.BlockSpec((BLK, feat), lambda i: (i, 0)),
        out_shape=jax.ShapeDtypeStruct((N, feat), jnp.float32),
    )(acc, hwp, dinv, b2d, Wres)


def kernel(x, edge_index, W1, b1, W2, W3, b3, W4, W5, b5, W6):
    src3 = edge_index[0].reshape(NW, NCHUNK, CH)
    dst3 = edge_index[1].reshape(NW, NCHUNK, CH)

    W3p = jnp.zeros((H1, 128), jnp.float32).at[:, :H2].set(W3)
    b3p = jnp.zeros((128,), jnp.float32).at[:H2].set(b3)
    W4p = jnp.zeros((128, 128), jnp.float32).at[:H2, :H2].set(W4)
    W5p = jnp.zeros((128, D_OUT), jnp.float32).at[:H2, :].set(W5)

    degp = _sc_degree(dst3)[:, :N].T
    hw1p, dinv = _tc_pre(x, W1, degp)

    acc1 = _sc_scatter(hw1p, src3, dst3, H1)[:, :N]
    hw2p = _tc_post(acc1, hw1p, dinv, b1.reshape(1, H1), W2, W3p)

    acc2 = _sc_scatter(hw2p, src3, dst3, 128)[:, :N]
    hw3p = _tc_post(acc2, hw2p, dinv, b3p.reshape(1, 128), W4p, W5p)

    acc3 = _sc_scatter(hw3p, src3, dst3, D_OUT)[:, :N]
    return _tc_final(acc3, hw3p, dinv, b5.reshape(1, D_OUT), W6)
